# baseline (speedup 1.0000x reference)
import functools

import jax
import jax.numpy as jnp
from jax import lax
from jax.experimental import pallas as pl
from jax.experimental.pallas import tpu as pltpu
from jax.experimental.pallas import tpu_sc as plsc

BATCH = 64
NUM_PATCHES = 576
PATCH_DIM = 768
PROJ_DIM = 96
NUM_MASK = 432
NUM_UNMASK = 144

NW = 32
U_TOT = BATCH * NUM_UNMASK
M_TOT = BATCH * NUM_MASK
U_PER_W = U_TOT // NW
M_PER_W = M_TOT // NW
CHUNK = 96
POS_PAD = 128

_sc_mesh = plsc.VectorSubcoreMesh(core_axis_name="c", subcore_axis_name="s")


@functools.partial(
    pl.kernel,
    out_type=(
        jax.ShapeDtypeStruct((U_TOT, PATCH_DIM), jnp.float32),
        jax.ShapeDtypeStruct((M_TOT, PROJ_DIM), jnp.float32),
        jax.ShapeDtypeStruct((U_TOT, PROJ_DIM), jnp.float32),
    ),
    mesh=_sc_mesh,
    scratch_types=[
        pltpu.VMEM((U_PER_W,), jnp.int32),
        pltpu.VMEM((U_PER_W,), jnp.int32),
        pltpu.VMEM((M_PER_W,), jnp.int32),
        pltpu.VMEM((CHUNK, PATCH_DIM), jnp.float32),
        pltpu.VMEM((CHUNK, PROJ_DIM), jnp.float32),
        pltpu.SemaphoreType.DMA,
    ],
    compiler_params=pltpu.CompilerParams(use_tc_tiling_on_sc=False),
)
def _sc_gather(patches_hbm, pos_plus_hbm, pos_hbm, uidx_hbm, midx_hbm,
               g_out, m_out, up_out,
               uidx_v, uidxg_v, midx_v, prow_v, rrow_v, sem):
    wid = lax.axis_index("s") * 2 + lax.axis_index("c")
    ubase = wid * U_PER_W
    mbase = wid * M_PER_W
    pltpu.sync_copy(uidx_hbm.at[pl.ds(ubase, U_PER_W)], uidx_v)
    pltpu.sync_copy(midx_hbm.at[pl.ds(mbase, M_PER_W)], midx_v)
    for i in range(U_PER_W // 16):
        boff = (2 * wid + (0 if i < NUM_UNMASK // 16 else 1)) * NUM_PATCHES
        uidxg_v[pl.ds(i * 16, 16)] = uidx_v[pl.ds(i * 16, 16)] + boff
    for c in range(U_PER_W // CHUNK):
        pltpu.async_copy(
            patches_hbm.at[uidxg_v.at[pl.ds(c * CHUNK, CHUNK)]], prow_v, sem
        ).wait()
        pltpu.sync_copy(prow_v, g_out.at[pl.ds(ubase + c * CHUNK, CHUNK)])
    for c in range(U_PER_W // CHUNK):
        pltpu.async_copy(
            pos_hbm.at[uidx_v.at[pl.ds(c * CHUNK, CHUNK)]], rrow_v, sem
        ).wait()
        pltpu.sync_copy(rrow_v, up_out.at[pl.ds(ubase + c * CHUNK, CHUNK)])
    for c in range(M_PER_W // CHUNK):
        pltpu.async_copy(
            pos_plus_hbm.at[midx_v.at[pl.ds(c * CHUNK, CHUNK)]], rrow_v, sem
        ).wait()
        pltpu.sync_copy(rrow_v, m_out.at[pl.ds(mbase + c * CHUNK, CHUNK)])


def _pos_plus_body(mt_ref, w_ref, b_ref, pos_ref, out_ref):
    mtw = jnp.dot(mt_ref[...], w_ref[...]) + b_ref[...]
    out_ref[...] = pos_ref[...] + mtw


def _proj_body(x_ref, w_ref, b_ref, o_ref):
    o_ref[...] = jnp.dot(x_ref[...], w_ref[...]) + b_ref[...]


_PROJ_BLOCK = 512


def kernel(patches, W, b, pos_table, mask_token):
    rand_indices = jnp.argsort(
        jax.random.uniform(jax.random.key(42), (BATCH, NUM_PATCHES)), axis=-1)
    mask_indices = rand_indices[:, :NUM_MASK]
    unmask_indices = rand_indices[:, NUM_MASK:]
    uidx_flat = unmask_indices.reshape(-1)
    midx_flat = mask_indices.reshape(-1)
    b2 = b.reshape(1, PROJ_DIM)

    pos_plus = pl.pallas_call(
        _pos_plus_body,
        out_shape=jax.ShapeDtypeStruct((NUM_PATCHES, PROJ_DIM), jnp.float32),
    )(mask_token, W, b2, pos_table)

    g_rows, m_rows, up_rows = _sc_gather(
        patches.reshape(BATCH * NUM_PATCHES, PATCH_DIM),
        pos_plus, pos_table, uidx_flat, midx_flat)

    unmasked_embeddings = pl.pallas_call(
        _proj_body,
        grid=(U_TOT // _PROJ_BLOCK,),
        in_specs=[
            pl.BlockSpec((_PROJ_BLOCK, PATCH_DIM), lambda i: (i, 0)),
            pl.BlockSpec((PATCH_DIM, PROJ_DIM), lambda i: (0, 0)),
            pl.BlockSpec((1, PROJ_DIM), lambda i: (0, 0)),
        ],
        out_specs=pl.BlockSpec((_PROJ_BLOCK, PROJ_DIM), lambda i: (i, 0)),
        out_shape=jax.ShapeDtypeStruct((U_TOT, PROJ_DIM), jnp.float32),
    )(g_rows, W, b2).reshape(BATCH, NUM_UNMASK, PROJ_DIM)

    return (
        unmasked_embeddings,
        m_rows.reshape(BATCH, NUM_MASK, PROJ_DIM),
        up_rows.reshape(BATCH, NUM_UNMASK, PROJ_DIM),
        mask_indices,
        unmask_indices,
    )


# speedup vs baseline: 1.9026x; 1.9026x over previous
import functools

import jax
import jax.numpy as jnp
from jax import lax
from jax.experimental import pallas as pl
from jax.experimental.pallas import tpu as pltpu
from jax.experimental.pallas import tpu_sc as plsc

BATCH = 64
NUM_PATCHES = 576
PATCH_DIM = 768
PROJ_DIM = 96
NUM_MASK = 432
NUM_UNMASK = 144

NW = 32
U_TOT = BATCH * NUM_UNMASK
M_TOT = BATCH * NUM_MASK
U_PER_W = U_TOT // NW
M_PER_W = M_TOT // NW
CHUNK = 96
POS_PAD = 128

_sc_mesh = plsc.VectorSubcoreMesh(core_axis_name="c", subcore_axis_name="s")


@functools.partial(
    pl.kernel,
    out_type=(
        jax.ShapeDtypeStruct((U_TOT, PATCH_DIM), jnp.float32),
        jax.ShapeDtypeStruct((M_TOT, POS_PAD), jnp.float32),
        jax.ShapeDtypeStruct((U_TOT, POS_PAD), jnp.float32),
    ),
    mesh=_sc_mesh,
    scratch_types=[
        pltpu.VMEM((U_PER_W,), jnp.int32),
        pltpu.VMEM((U_PER_W,), jnp.int32),
        pltpu.VMEM((M_PER_W,), jnp.int32),
        pltpu.VMEM((CHUNK, PATCH_DIM), jnp.float32),
        pltpu.VMEM((CHUNK, POS_PAD), jnp.float32),
        pltpu.SemaphoreType.DMA,
    ],
)
def _sc_gather(patches_hbm, pos_plus_hbm, pos_hbm, uidx_hbm, midx_hbm,
               g_out, m_out, up_out,
               uidx_v, uidxg_v, midx_v, prow_v, rrow_v, sem):
    wid = lax.axis_index("s") * 2 + lax.axis_index("c")
    ubase = wid * U_PER_W
    mbase = wid * M_PER_W
    pltpu.sync_copy(uidx_hbm.at[pl.ds(ubase, U_PER_W)], uidx_v)
    pltpu.sync_copy(midx_hbm.at[pl.ds(mbase, M_PER_W)], midx_v)
    for i in range(U_PER_W // 16):
        boff = (2 * wid + (0 if i < NUM_UNMASK // 16 else 1)) * NUM_PATCHES
        uidxg_v[pl.ds(i * 16, 16)] = uidx_v[pl.ds(i * 16, 16)] + boff
    for c in range(U_PER_W // CHUNK):
        pltpu.async_copy(
            patches_hbm.at[uidxg_v.at[pl.ds(c * CHUNK, CHUNK)]], prow_v, sem
        ).wait()
        pltpu.sync_copy(prow_v, g_out.at[pl.ds(ubase + c * CHUNK, CHUNK)])
    for c in range(U_PER_W // CHUNK):
        pltpu.async_copy(
            pos_hbm.at[uidx_v.at[pl.ds(c * CHUNK, CHUNK)]], rrow_v, sem
        ).wait()
        pltpu.sync_copy(rrow_v, up_out.at[pl.ds(ubase + c * CHUNK, CHUNK)])
    for c in range(M_PER_W // CHUNK):
        pltpu.async_copy(
            pos_plus_hbm.at[midx_v.at[pl.ds(c * CHUNK, CHUNK)]], rrow_v, sem
        ).wait()
        pltpu.sync_copy(rrow_v, m_out.at[pl.ds(mbase + c * CHUNK, CHUNK)])


def _pos_plus_body(mt_ref, w_ref, b_ref, pos_ref, out_ref):
    mtw = jnp.dot(mt_ref[...], w_ref[...]) + b_ref[...]
    out_ref[...] = pos_ref[...] + mtw


def _proj_body(x_ref, w_ref, b_ref, mpad_ref, uppad_ref,
               o_ref, m_ref, up_ref):
    o_ref[...] = jnp.dot(x_ref[...], w_ref[...]) + b_ref[...]
    m_ref[...] = mpad_ref[:, :PROJ_DIM]
    up_ref[...] = uppad_ref[:, :PROJ_DIM]


_NSTEP = 18
_UBLK = U_TOT // _NSTEP
_MBLK = M_TOT // _NSTEP


def kernel(patches, W, b, pos_table, mask_token):
    rand_indices = jnp.argsort(
        jax.random.uniform(jax.random.key(42), (BATCH, NUM_PATCHES)), axis=-1)
    mask_indices = rand_indices[:, :NUM_MASK]
    unmask_indices = rand_indices[:, NUM_MASK:]
    uidx_flat = unmask_indices.reshape(-1)
    midx_flat = mask_indices.reshape(-1)
    b2 = b.reshape(1, PROJ_DIM)
    w_pad = jnp.pad(W, ((0, 0), (0, POS_PAD - PROJ_DIM)))
    b_pad = jnp.pad(b2, ((0, 0), (0, POS_PAD - PROJ_DIM)))
    pos_pad = jnp.pad(pos_table, ((0, 0), (0, POS_PAD - PROJ_DIM)))

    pos_plus = pl.pallas_call(
        _pos_plus_body,
        out_shape=jax.ShapeDtypeStruct((NUM_PATCHES, POS_PAD), jnp.float32),
    )(mask_token, w_pad, b_pad, pos_pad)

    g_rows, m_pad_rows, up_pad_rows = _sc_gather(
        patches.reshape(BATCH * NUM_PATCHES, PATCH_DIM),
        pos_plus, pos_pad, uidx_flat, midx_flat)

    ue, m_rows, up_rows = pl.pallas_call(
        _proj_body,
        grid=(_NSTEP,),
        in_specs=[
            pl.BlockSpec((_UBLK, PATCH_DIM), lambda i: (i, 0)),
            pl.BlockSpec((PATCH_DIM, PROJ_DIM), lambda i: (0, 0)),
            pl.BlockSpec((1, PROJ_DIM), lambda i: (0, 0)),
            pl.BlockSpec((_MBLK, POS_PAD), lambda i: (i, 0)),
            pl.BlockSpec((_UBLK, POS_PAD), lambda i: (i, 0)),
        ],
        out_specs=[
            pl.BlockSpec((_UBLK, PROJ_DIM), lambda i: (i, 0)),
            pl.BlockSpec((_MBLK, PROJ_DIM), lambda i: (i, 0)),
            pl.BlockSpec((_UBLK, PROJ_DIM), lambda i: (i, 0)),
        ],
        out_shape=[
            jax.ShapeDtypeStruct((U_TOT, PROJ_DIM), jnp.float32),
            jax.ShapeDtypeStruct((M_TOT, PROJ_DIM), jnp.float32),
            jax.ShapeDtypeStruct((U_TOT, PROJ_DIM), jnp.float32),
        ],
    )(g_rows, W, b2, m_pad_rows, up_pad_rows)

    return (
        ue.reshape(BATCH, NUM_UNMASK, PROJ_DIM),
        m_rows.reshape(BATCH, NUM_MASK, PROJ_DIM),
        up_rows.reshape(BATCH, NUM_UNMASK, PROJ_DIM),
        mask_indices,
        unmask_indices,
    )
